# baseline (device time: 84267 ns/iter reference)
import jax
import jax.numpy as jnp
from jax import lax
from jax.experimental import pallas as pl
from jax.experimental.pallas import tpu as pltpu

N_DEV = 4
B = 16
H = 16
D = 64
BS = 16
NB = 128
P_LOCAL = 128
NKEYS = P_LOCAL * BS
NEG = -1e30


def _body(q_ref, k_ref, v_ref, cnt_ref, out_ref,
          mine_ref, comm_ref, send_sems, recv_sems):
    my = lax.axis_index("i")

    bsem = pltpu.get_barrier_semaphore()
    for k in (1, 2, 3):
        pl.semaphore_signal(
            bsem, inc=1,
            device_id=((my + k) % N_DEV,),
            device_id_type=pl.DeviceIdType.MESH,
        )
    pl.semaphore_wait(bsem, N_DEV - 1)

    cnt = cnt_ref[:, :]
    scale = D ** -0.5
    for h in range(H):
        q_h = q_ref[:, h, :].astype(jnp.bfloat16)
        k_h = k_ref[:, :, h, :].reshape(NKEYS, D).astype(jnp.bfloat16)
        s = lax.dot_general(
            q_h, k_h, (((1,), (1,)), ((), ())),
            preferred_element_type=jnp.float32,
        ) * scale
        s = jnp.where(cnt > 0.0, s, NEG)
        m = jnp.max(s, axis=1, keepdims=True)
        p = jnp.exp(s - m) * cnt
        l = jnp.sum(p, axis=1, keepdims=True)
        v_h = v_ref[:, :, h, :].reshape(NKEYS, D).astype(jnp.bfloat16)
        o = lax.dot_general(
            p.astype(jnp.bfloat16), v_h, (((1,), (0,)), ((), ())),
            preferred_element_type=jnp.float32,
        )
        mine_ref[:, h, 0:D] = o
        mine_ref[:, h, D:D + 1] = m
        mine_ref[:, h, D + 1:D + 2] = l

    descs = []
    for k in (1, 2, 3):
        r = 3 - k
        rdma = pltpu.make_async_remote_copy(
            src_ref=mine_ref,
            dst_ref=comm_ref.at[r],
            send_sem=send_sems.at[k - 1],
            recv_sem=recv_sems.at[r],
            device_id=((my + k) % N_DEV,),
            device_id_type=pl.DeviceIdType.MESH,
        )
        rdma.start()
        descs.append(rdma)
    for d in descs:
        d.wait_send()
    for d in descs:
        d.wait_recv()

    mine = mine_ref[:, :, :]
    m_g = mine[:, :, D:D + 1]
    for r in range(3):
        m_g = jnp.maximum(m_g, comm_ref[r, :, :, D:D + 1])
    acc = mine * jnp.exp(mine[:, :, D:D + 1] - m_g)
    for r in range(3):
        part = comm_ref[r, :, :, :]
        acc = acc + part * jnp.exp(part[:, :, D:D + 1] - m_g)
    out_ref[:, 0, :, :] = acc[:, :, 0:D] / acc[:, :, D + 1:D + 2]


def kernel(Q, K, V, bt, lens):
    my = lax.axis_index("i")
    off = my * P_LOCAL

    ids = off + jnp.arange(P_LOCAL, dtype=jnp.int32)
    valid = jnp.arange(NB, dtype=jnp.int32)[None, :] < lens[:, None]
    eq = bt[:, :, None] == ids[None, None, :]
    counts = jnp.sum(
        jnp.where(eq & valid[:, :, None], 1.0, 0.0), axis=1
    ).astype(jnp.float32)
    cnt_keys = jnp.repeat(counts, BS, axis=1)
    qs = Q[:, 0]

    return pl.pallas_call(
        _body,
        out_shape=jax.ShapeDtypeStruct((B, 1, H, D), jnp.float32),
        in_specs=[pl.BlockSpec(memory_space=pltpu.VMEM)] * 4,
        out_specs=pl.BlockSpec(memory_space=pltpu.VMEM),
        scratch_shapes=[
            pltpu.VMEM((B, H, 128), jnp.float32),
            pltpu.VMEM((3, B, H, 128), jnp.float32),
            pltpu.SemaphoreType.DMA((3,)),
            pltpu.SemaphoreType.DMA((3,)),
        ],
        compiler_params=pltpu.CompilerParams(collective_id=0),
    )(qs, K, V, cnt_keys)


# device time: 35969 ns/iter; 2.3428x vs baseline; 2.3428x over previous
import jax
import jax.numpy as jnp
from jax import lax
from jax.experimental import pallas as pl
from jax.experimental.pallas import tpu as pltpu

N_DEV = 4
B = 16
H = 16
D = 64
BS = 16
NB = 128
P_LOCAL = 128
NKEYS = P_LOCAL * BS
NEG = -1e30


def _body(q_ref, k_ref, v_ref, cnt_ref, out_ref,
          mine_ref, comm_ref, send_sems, recv_sems):
    my = lax.axis_index("i")

    bsem = pltpu.get_barrier_semaphore()
    for k in (1, 2, 3):
        pl.semaphore_signal(
            bsem, inc=1,
            device_id=((my + k) % N_DEV,),
            device_id_type=pl.DeviceIdType.MESH,
        )
    pl.semaphore_wait(bsem, N_DEV - 1)

    cnt = cnt_ref[:, :]
    scale = D ** -0.5
    for h in range(H):
        q_h = q_ref[h, :, :]
        k_h = k_ref[h, :, :]
        s = lax.dot_general(
            q_h, k_h, (((1,), (1,)), ((), ())),
            preferred_element_type=jnp.float32,
        ) * scale
        s = jnp.where(cnt > 0.0, s, NEG)
        m = jnp.max(s, axis=1, keepdims=True)
        p = jnp.exp(s - m) * cnt
        l = jnp.sum(p, axis=1, keepdims=True)
        v_h = v_ref[h, :, :]
        o = lax.dot_general(
            p.astype(jnp.bfloat16), v_h, (((1,), (0,)), ((), ())),
            preferred_element_type=jnp.float32,
        )
        mine_ref[:, h, 0:D] = o
        mine_ref[:, h, D:D + 1] = m
        mine_ref[:, h, D + 1:D + 2] = l

    descs = []
    for k in (1, 2, 3):
        r = 3 - k
        rdma = pltpu.make_async_remote_copy(
            src_ref=mine_ref,
            dst_ref=comm_ref.at[r],
            send_sem=send_sems.at[k - 1],
            recv_sem=recv_sems.at[r],
            device_id=((my + k) % N_DEV,),
            device_id_type=pl.DeviceIdType.MESH,
        )
        rdma.start()
        descs.append(rdma)
    for d in descs:
        d.wait_send()
    for d in descs:
        d.wait_recv()

    mine = mine_ref[:, :, :]
    m_g = mine[:, :, D:D + 1]
    for r in range(3):
        m_g = jnp.maximum(m_g, comm_ref[r, :, :, D:D + 1])
    acc = mine * jnp.exp(mine[:, :, D:D + 1] - m_g)
    for r in range(3):
        part = comm_ref[r, :, :, :]
        acc = acc + part * jnp.exp(part[:, :, D:D + 1] - m_g)
    out_ref[:, 0, :, :] = acc[:, :, 0:D] / acc[:, :, D + 1:D + 2]


def kernel(Q, K, V, bt, lens):
    my = lax.axis_index("i")
    off = my * P_LOCAL

    ids = off + jnp.arange(P_LOCAL, dtype=jnp.int32)
    valid = jnp.arange(NB, dtype=jnp.int32)[None, :] < lens[:, None]
    eq = bt[:, :, None] == ids[None, None, :]
    counts = jnp.sum(
        jnp.where(eq & valid[:, :, None], 1.0, 0.0), axis=1
    ).astype(jnp.float32)
    cnt_keys = jnp.repeat(counts, BS, axis=1)
    qs = Q[:, 0].swapaxes(0, 1).astype(jnp.bfloat16)
    kt = K.reshape(NKEYS, H, D).swapaxes(0, 1).astype(jnp.bfloat16)
    vt = V.reshape(NKEYS, H, D).swapaxes(0, 1).astype(jnp.bfloat16)

    return pl.pallas_call(
        _body,
        out_shape=jax.ShapeDtypeStruct((B, 1, H, D), jnp.float32),
        in_specs=[pl.BlockSpec(memory_space=pltpu.VMEM)] * 4,
        out_specs=pl.BlockSpec(memory_space=pltpu.VMEM),
        scratch_shapes=[
            pltpu.VMEM((B, H, 128), jnp.float32),
            pltpu.VMEM((3, B, H, 128), jnp.float32),
            pltpu.SemaphoreType.DMA((3,)),
            pltpu.SemaphoreType.DMA((3,)),
        ],
        compiler_params=pltpu.CompilerParams(collective_id=0),
    )(qs, kt, vt, cnt_keys)
